# Initial kernel scaffold; baseline (speedup 1.0000x reference)
import jax
import jax.numpy as jnp
from jax.experimental import pallas as pl


def kernel(x, edge_index, edge_weight, W1, b1, W2, b2, W3, b3, Wmu, bmu, Wls, bls):
    raise NotImplementedError("write your pallas kernel here")


# speedup vs baseline: 8.1424x; 8.1424x over previous
import functools

import jax
import jax.numpy as jnp
from jax import lax
from jax.experimental import pallas as pl
from jax.experimental.pallas import tpu as pltpu
from jax.experimental.pallas import tpu_sc as plsc

N = 10000
E = 320000
D = 128

NC = 2
NS = 16
E_PER_CORE = E // NC
E_PER_TILE = E_PER_CORE // NS
K = 80
NCHUNK = E_PER_TILE // K
ROWS_PER_TILE = N // NS
NPAD = 10240
DPT = NPAD // NS


def _mesh():
    return plsc.VectorSubcoreMesh(
        core_axis_name="c", subcore_axis_name="s", num_cores=NC, num_subcores=NS
    )


DK = 80


@functools.partial(
    pl.kernel,
    out_type=jax.ShapeDtypeStruct((NC, 1, NPAD), jnp.float32),
    mesh=_mesh(),
    scratch_types=dict(
        cidx=pltpu.VMEM((DK,), jnp.int32),
        wv=pltpu.VMEM((DK,), jnp.float32),
        zb=pltpu.VMEM((DPT,), jnp.float32),
        acc=pltpu.VMEM_SHARED((NPAD,), jnp.float32),
    ),
)
def _deg_sc(col_hbm, w_hbm, out_hbm, cidx, wv, zb, acc):
    cid = lax.axis_index("c")
    sid = lax.axis_index("s")

    zero16 = jnp.zeros((16,), jnp.float32)

    def zloop(i, _):
        zb[pl.ds(i * 16, 16)] = zero16
        return ()

    lax.fori_loop(0, DPT // 16, zloop, ())
    pltpu.sync_copy(zb, acc.at[pl.ds(sid * DPT, DPT)])
    plsc.subcore_barrier()

    ebase = cid * E_PER_CORE + sid * E_PER_TILE

    def body(i, _):
        off = ebase + i * DK
        pltpu.sync_copy(col_hbm.at[pl.ds(off, DK)], cidx)
        pltpu.sync_copy(w_hbm.at[pl.ds(off, DK)], wv)
        pltpu.sync_copy(wv, acc.at[cidx], add=True)
        return ()

    lax.fori_loop(0, E_PER_TILE // DK, body, ())
    plsc.subcore_barrier()
    pltpu.sync_copy(acc.at[pl.ds(sid * DPT, DPT)], zb)
    pltpu.sync_copy(zb, out_hbm.at[cid, 0, pl.ds(sid * DPT, DPT)])


ZROWS = 128


@functools.partial(
    pl.kernel,
    out_type=jax.ShapeDtypeStruct((NC, NPAD, D), jnp.float32),
    mesh=_mesh(),
    scratch_types=dict(
        ridx=pltpu.VMEM((K,), jnp.int32),
        cidx=pltpu.VMEM((K,), jnp.int32),
        wv=pltpu.VMEM((K,), jnp.float32),
        gbuf=pltpu.VMEM((K, D), jnp.float32),
        sbuf=pltpu.VMEM((K, D), jnp.float32),
        zb=pltpu.VMEM((ZROWS, D), jnp.float32),
        acc=pltpu.VMEM_SHARED((NPAD, D), jnp.float32),
        sem=pltpu.SemaphoreType.DMA,
    ),
)
def _prop_sc(hs_hbm, row_hbm, col_hbm, w_hbm, out_hbm, ridx, cidx, wv, gbuf,
             sbuf, zb, acc, sem):
    cid = lax.axis_index("c")
    sid = lax.axis_index("s")

    zero16 = jnp.zeros((16,), jnp.float32)

    def zloop(i, _):
        r = i // (D // 16)
        f = i % (D // 16)
        zb[r, pl.ds(f * 16, 16)] = zero16
        return ()

    lax.fori_loop(0, ZROWS * (D // 16), zloop, ())

    def minit(i, _):
        pltpu.sync_copy(zb, acc.at[pl.ds(sid * DPT + i * ZROWS, ZROWS)])
        return ()

    lax.fori_loop(0, DPT // ZROWS, minit, ())
    plsc.subcore_barrier()

    ebase = cid * E_PER_CORE + sid * E_PER_TILE

    def body(i, _):
        off = ebase + i * K
        pltpu.sync_copy(row_hbm.at[pl.ds(off, K)], ridx)
        pltpu.sync_copy(col_hbm.at[pl.ds(off, K)], cidx)
        pltpu.sync_copy(w_hbm.at[pl.ds(off, K)], wv)
        pltpu.async_copy(hs_hbm.at[ridx], gbuf, sem).wait()

        def scale(g, _):
            wg = wv[pl.ds(g * 16, 16)]
            for l in range(16):
                j = g * 16 + l
                s = wg[l]
                for f in range(D // 16):
                    sbuf[j, pl.ds(f * 16, 16)] = gbuf[j, pl.ds(f * 16, 16)] * s
            return ()

        lax.fori_loop(0, K // 16, scale, ())
        pltpu.sync_copy(sbuf, acc.at[cidx], add=True)
        return ()

    lax.fori_loop(0, NCHUNK, body, ())
    plsc.subcore_barrier()

    def cpout(i, _):
        r0 = sid * DPT + i * ZROWS
        pltpu.sync_copy(acc.at[pl.ds(r0, ZROWS)], zb)
        pltpu.sync_copy(zb, out_hbm.at[cid, pl.ds(r0, ZROWS)])
        return ()

    lax.fori_loop(0, DPT // ZROWS, cpout, ())


R = 400
GRID = N // R


def _rows(i):
    return (i, 0)


def _full(i):
    return (0, 0)


def _vspec(cols):
    return pl.BlockSpec((R, cols), _rows)


def _wspec(r, c):
    return pl.BlockSpec((r, c), _full)


def _t0_body(deg_ref, x_ref, dis_ref, hs_ref):
    deg = deg_ref[...] + 1.0
    dis = lax.rsqrt(deg)
    dis_ref[...] = dis
    hs_ref[...] = dis * x_ref[...]


def _t0(deg, x):
    return pl.pallas_call(
        _t0_body,
        grid=(GRID,),
        in_specs=[pl.BlockSpec((R, 1), _rows), _vspec(D)],
        out_specs=[pl.BlockSpec((R, 1), _rows), _vspec(D)],
        out_shape=[
            jax.ShapeDtypeStruct((N, 1), jnp.float32),
            jax.ShapeDtypeStruct((N, D), jnp.float32),
        ],
    )(deg, x)


def _mid_body(act, a0_ref, a1_ref, hs_ref, dis_ref, w_ref, b_ref, *out_refs):
    dis = dis_ref[...]
    p = dis * (a0_ref[...] + a1_ref[...] + hs_ref[...])
    z = jnp.dot(p, w_ref[...], preferred_element_type=jnp.float32) + b_ref[...]
    z = act(z)
    hs = dis * z
    if len(out_refs) == 1:
        out_refs[0][...] = hs
    else:
        h = hs.shape[1] // 2
        out_refs[0][...] = hs[:, :h]
        out_refs[1][...] = hs[:, h:]


def _t_mid(act, a, hs, dis, w, b, split):
    cin = hs.shape[1]
    cout = w.shape[1]
    if split:
        out_specs = [_vspec(cout // 2), _vspec(cout // 2)]
        out_shape = [jax.ShapeDtypeStruct((N, cout // 2), jnp.float32)] * 2
    else:
        out_specs = [_vspec(cout)]
        out_shape = [jax.ShapeDtypeStruct((N, cout), jnp.float32)]
    return pl.pallas_call(
        functools.partial(_mid_body, act),
        grid=(GRID,),
        in_specs=[
            _vspec(cin), _vspec(cin), _vspec(cin),
            pl.BlockSpec((R, 1), _rows),
            _wspec(cin, cout), _wspec(1, cout),
        ],
        out_specs=out_specs,
        out_shape=out_shape,
    )(a[0], a[1], hs, dis, w, b)


def _t3_body(a30_ref, a31_ref, b30_ref, b31_ref, hsa_ref, hsb_ref, dis_ref,
             w3_ref, b3_ref, wc_ref, hs4_ref):
    dis = dis_ref[...]
    pa = dis * (a30_ref[...] + a31_ref[...] + hsa_ref[...])
    pb = dis * (b30_ref[...] + b31_ref[...] + hsb_ref[...])
    p = jnp.concatenate([pa, pb], axis=1)
    z3 = jnp.dot(p, w3_ref[...], preferred_element_type=jnp.float32) + b3_ref[...]
    z3 = jnp.maximum(z3, 0.0)
    c4 = jnp.dot(z3, wc_ref[...], preferred_element_type=jnp.float32)
    hs4_ref[...] = dis * c4


def _t3(a3a, a3b, hsa, hsb, dis, w3, b3, wc):
    return pl.pallas_call(
        _t3_body,
        grid=(GRID,),
        in_specs=[
            _vspec(D), _vspec(D), _vspec(D), _vspec(D), _vspec(D), _vspec(D),
            pl.BlockSpec((R, 1), _rows),
            _wspec(256, 512), _wspec(1, 512), _wspec(512, D),
        ],
        out_specs=[_vspec(D)],
        out_shape=[jax.ShapeDtypeStruct((N, D), jnp.float32)],
    )(a3a[0], a3a[1], a3b[0], a3b[1], hsa, hsb, dis, w3, b3, wc)


def _t4_body(a0_ref, a1_ref, hs_ref, dis_ref, bmu_ref, bls_ref, mu_ref, ls_ref):
    dis = dis_ref[...]
    p = dis * (a0_ref[...] + a1_ref[...] + hs_ref[...])
    mu_ref[...] = p[:, :64] + bmu_ref[...]
    ls_ref[...] = p[:, 64:] + bls_ref[...]


def _t4(a, hs, dis, bmu, bls):
    return pl.pallas_call(
        _t4_body,
        grid=(GRID,),
        in_specs=[
            _vspec(D), _vspec(D), _vspec(D),
            pl.BlockSpec((R, 1), _rows),
            _wspec(1, 64), _wspec(1, 64),
        ],
        out_specs=[_vspec(64), _vspec(64)],
        out_shape=[jax.ShapeDtypeStruct((N, 64), jnp.float32)] * 2,
    )(a[0], a[1], hs, dis, bmu, bls)


def kernel(x, edge_index, edge_weight, W1, b1, W2, b2, W3, b3, Wmu, bmu, Wls, bls):
    row = edge_index[0].astype(jnp.int32)
    col = edge_index[1].astype(jnp.int32)
    w = edge_weight.astype(jnp.float32)

    degh = _deg_sc(col, w)
    deg = (degh[0, 0, :N] + degh[1, 0, :N]).reshape(N, 1)

    dis, hs0 = _t0(deg, x)

    b1r = b1.reshape(1, -1)
    b2r = b2.reshape(1, -1)
    b3r = b3.reshape(1, -1)
    wc = jnp.concatenate([Wmu, Wls], axis=1)

    def prop(hs):
        a = _prop_sc(hs, row, col, w)
        return a[:, :N, :]

    a1 = prop(hs0)
    (hs1,) = _t_mid(jax.nn.sigmoid, a1, hs0, dis, W1, b1r, split=False)

    a2 = prop(hs1)
    hs2a, hs2b = _t_mid(lambda z: jnp.maximum(z, 0.0), a2, hs1, dis, W2, b2r,
                        split=True)

    a3a = prop(hs2a)
    a3b = prop(hs2b)
    (hs4,) = _t3(a3a, a3b, hs2a, hs2b, dis, W3, b3r, wc)

    a4 = prop(hs4)
    mu, ls = _t4(a4, hs4, dis, bmu.reshape(1, -1), bls.reshape(1, -1))
    return (mu, ls)
